# baseline (speedup 1.0000x reference)
import jax
import jax.numpy as jnp
from jax.experimental import pallas as pl


def _relu_block(x_ref, o_ref):
    o_ref[...] = jnp.maximum(x_ref[...], 0.0)


def kernel(x):
    n, h, w, c = x.shape
    xt = x.transpose(0, 1, 3, 2).reshape(n * h, c, w)
    BR = 28
    out = pl.pallas_call(
        _relu_block,
        grid=(n * h // BR,),
        in_specs=[pl.BlockSpec((BR, c, w), lambda i: (i, 0, 0))],
        out_specs=pl.BlockSpec((BR, c, w), lambda i: (i, 0, 0)),
        out_shape=jax.ShapeDtypeStruct((n * h, c, w), x.dtype),
    )(xt)
    return out.reshape(n, h, c, w).transpose(0, 1, 3, 2)


# speedup vs baseline: 1.0470x; 1.0470x over previous
import jax
import jax.numpy as jnp
from jax.experimental import pallas as pl


def _relu_block(x_ref, o_ref):
    o_ref[...] = jnp.maximum(x_ref[...], 0.0)


def kernel(x):
    n, h, w, c = x.shape
    xt = x.transpose(0, 1, 3, 2).reshape(n * h, c, w)
    BR = 112
    out = pl.pallas_call(
        _relu_block,
        grid=(n * h // BR,),
        in_specs=[pl.BlockSpec((BR, c, w), lambda i: (i, 0, 0))],
        out_specs=pl.BlockSpec((BR, c, w), lambda i: (i, 0, 0)),
        out_shape=jax.ShapeDtypeStruct((n * h, c, w), x.dtype),
    )(xt)
    return out.reshape(n, h, c, w).transpose(0, 1, 3, 2)


# speedup vs baseline: 1.0484x; 1.0014x over previous
import jax
import jax.numpy as jnp
from jax.experimental import pallas as pl


def _relu_block(x_ref, o_ref):
    o_ref[...] = jnp.maximum(x_ref[...], 0.0)


def kernel(x):
    n, h, w, c = x.shape
    xt = x.transpose(0, 1, 3, 2).reshape(n * h, c, w)
    BR = 128
    out = pl.pallas_call(
        _relu_block,
        grid=(n * h // BR,),
        in_specs=[pl.BlockSpec((BR, c, w), lambda i: (i, 0, 0))],
        out_specs=pl.BlockSpec((BR, c, w), lambda i: (i, 0, 0)),
        out_shape=jax.ShapeDtypeStruct((n * h, c, w), x.dtype),
    )(xt)
    return out.reshape(n, h, c, w).transpose(0, 1, 3, 2)
